# Initial kernel scaffold; baseline (speedup 1.0000x reference)
import functools

import jax
import jax.numpy as jnp
from jax import lax
from jax.experimental import pallas as pl
from jax.experimental.pallas import tpu as pltpu
from jax.experimental.pallas import tpu_sc as plsc

BUCKETS_C = 1000000
HASH_DIM_C = 32
MODEL_DIM_C = 128


def _hash_body(ids_ref, prev_ref, out_ref):
    a = ids_ref[...].astype(jnp.uint32)
    p = prev_ref[...].astype(jnp.uint32)
    m = jnp.uint32(BUCKETS_C)
    t1 = (a % m) * jnp.uint32(8000) % m
    t2 = (a % m) * jnp.uint32(191) % m
    h = ((t1 + t2) % m + p % m) % m
    out_ref[...] = h.astype(jnp.int32)


def _hash_tc(ids2d, prev2d):
    return pl.pallas_call(
        _hash_body,
        out_shape=jax.ShapeDtypeStruct(ids2d.shape, jnp.int32),
    )(ids2d, prev2d)


def _gather_sc(hashed, table):
    n = hashed.shape[0]
    d = table.shape[1]
    info = plsc.get_sparse_core_info()
    nc, ns = info.num_cores, info.num_subcores
    nw = nc * ns
    per_w = n // nw
    out_ch = 1024
    n_outer = per_w // out_ch
    ng = out_ch // 128

    mesh = plsc.VectorSubcoreMesh(core_axis_name="c", subcore_axis_name="s")

    @functools.partial(
        pl.kernel,
        out_type=jax.ShapeDtypeStruct((n, d), jnp.float32),
        mesh=mesh,
        scratch_types=[
            pltpu.VMEM((out_ch,), jnp.int32),
            pltpu.VMEM((out_ch, d), jnp.float32),
            pltpu.SemaphoreType.DMA,
        ],
    )
    def k(hashed_hbm, table_hbm, out_hbm, idx_v, rows_v, sem):
        wid = lax.axis_index("s") * nc + lax.axis_index("c")
        base = wid * per_w

        def body(c, carry):
            off = base + c * out_ch
            pltpu.sync_copy(hashed_hbm.at[pl.ds(off, out_ch)], idx_v)
            cps = [
                pltpu.async_copy(
                    table_hbm.at[idx_v.at[pl.ds(j * 128, 128)]],
                    rows_v.at[pl.ds(j * 128, 128), :],
                    sem,
                )
                for j in range(ng)
            ]
            for cp in cps:
                cp.wait()
            pltpu.sync_copy(rows_v, out_hbm.at[pl.ds(off, out_ch)])
            return carry

        lax.fori_loop(0, n_outer, body, 0)

    return k(hashed, table)


def _mm_body(emb_ref, wt_ref, b_ref, out_ref):
    out_ref[...] = (
        jnp.dot(emb_ref[...], wt_ref[...], preferred_element_type=jnp.float32)
        + b_ref[...]
    )


def _matmul_tc(emb, wt, b2d):
    n, d = emb.shape
    bm = 2048
    grid = n // bm
    return pl.pallas_call(
        _mm_body,
        grid=(grid,),
        in_specs=[
            pl.BlockSpec((bm, d), lambda i: (i, 0)),
            pl.BlockSpec((d, MODEL_DIM_C), lambda i: (0, 0)),
            pl.BlockSpec((1, MODEL_DIM_C), lambda i: (0, 0)),
        ],
        out_specs=pl.BlockSpec((bm, MODEL_DIM_C), lambda i: (i, 0)),
        out_shape=jax.ShapeDtypeStruct((n, MODEL_DIM_C), jnp.float32),
    )(emb, wt, b2d)


def kernel(input_ids, table, W, b):
    bsz, seq = input_ids.shape
    n = bsz * seq
    ids_flat = input_ids.reshape(n)
    prev_flat = jnp.pad(input_ids[:, :-1], ((0, 0), (1, 0))).reshape(n)

    hashed = _hash_tc(
        ids_flat.reshape(n // 128, 128), prev_flat.reshape(n // 128, 128)
    ).reshape(n)
    emb = _gather_sc(hashed, table)
    out = _matmul_tc(emb, W.T, b.reshape(1, MODEL_DIM_C))
    return out.reshape(bsz, seq, MODEL_DIM_C)


# speedup vs baseline: 16.0409x; 16.0409x over previous
import functools

import jax
import jax.numpy as jnp
from jax import lax
from jax.experimental import pallas as pl
from jax.experimental.pallas import tpu as pltpu
from jax.experimental.pallas import tpu_sc as plsc

BUCKETS_C = 1000000
HASH_DIM_C = 32
MODEL_DIM_C = 128


def _hash_body(ids_ref, prev_ref, out_ref):
    a = ids_ref[...].astype(jnp.uint32)
    p = prev_ref[...].astype(jnp.uint32)
    m = jnp.uint32(BUCKETS_C)
    t1 = (a % m) * jnp.uint32(8000) % m
    t2 = (a % m) * jnp.uint32(191) % m
    h = ((t1 + t2) % m + p % m) % m
    out_ref[...] = h.astype(jnp.int32)


def _hash_tc(ids2d, prev2d):
    return pl.pallas_call(
        _hash_body,
        out_shape=jax.ShapeDtypeStruct(ids2d.shape, jnp.int32),
    )(ids2d, prev2d)


def _gather_sc(hashed, table):
    n = hashed.shape[0]
    d = table.shape[1]
    info = plsc.get_sparse_core_info()
    nc, ns = info.num_cores, info.num_subcores
    nw = nc * ns
    per_w = n // nw
    out_ch = 1024
    n_outer = per_w // out_ch
    ng = out_ch // 128

    mesh = plsc.VectorSubcoreMesh(core_axis_name="c", subcore_axis_name="s")

    @functools.partial(
        pl.kernel,
        out_type=jax.ShapeDtypeStruct((n, d), jnp.float32),
        mesh=mesh,
        scratch_types=[
            pltpu.VMEM((out_ch,), jnp.int32),
            pltpu.VMEM((out_ch, d), jnp.float32),
            pltpu.SemaphoreType.DMA,
        ],
        compiler_params=pltpu.CompilerParams(use_tc_tiling_on_sc=False),
    )
    def k(hashed_hbm, table_hbm, out_hbm, idx_v, rows_v, sem):
        wid = lax.axis_index("s") * nc + lax.axis_index("c")
        base = wid * per_w

        def body(c, carry):
            off = base + c * out_ch
            pltpu.sync_copy(hashed_hbm.at[pl.ds(off, out_ch)], idx_v)
            cps = [
                pltpu.async_copy(
                    table_hbm.at[idx_v.at[pl.ds(j * 128, 128)]],
                    rows_v.at[pl.ds(j * 128, 128), :],
                    sem,
                )
                for j in range(ng)
            ]
            for cp in cps:
                cp.wait()
            pltpu.sync_copy(rows_v, out_hbm.at[pl.ds(off, out_ch)])
            return carry

        lax.fori_loop(0, n_outer, body, 0)

    return k(hashed, table)


def _mm_body(emb_ref, wt_ref, b_ref, out_ref):
    out_ref[...] = (
        jnp.dot(emb_ref[...], wt_ref[...], preferred_element_type=jnp.float32)
        + b_ref[...]
    )


def _matmul_tc(emb, wt, b2d):
    n, d = emb.shape
    bm = 2048
    grid = n // bm
    return pl.pallas_call(
        _mm_body,
        grid=(grid,),
        in_specs=[
            pl.BlockSpec((bm, d), lambda i: (i, 0)),
            pl.BlockSpec((d, MODEL_DIM_C), lambda i: (0, 0)),
            pl.BlockSpec((1, MODEL_DIM_C), lambda i: (0, 0)),
        ],
        out_specs=pl.BlockSpec((bm, MODEL_DIM_C), lambda i: (i, 0)),
        out_shape=jax.ShapeDtypeStruct((n, MODEL_DIM_C), jnp.float32),
    )(emb, wt, b2d)


def kernel(input_ids, table, W, b):
    bsz, seq = input_ids.shape
    n = bsz * seq
    ids_flat = input_ids.reshape(n)
    prev_flat = jnp.pad(input_ids[:, :-1], ((0, 0), (1, 0))).reshape(n)

    hashed = _hash_tc(
        ids_flat.reshape(n // 128, 128), prev_flat.reshape(n // 128, 128)
    ).reshape(n)
    emb = _gather_sc(hashed, table)
    out = _matmul_tc(emb, W.T, b.reshape(1, MODEL_DIM_C))
    return out.reshape(bsz, seq, MODEL_DIM_C)
